# baseline (speedup 1.0000x reference)
import functools

import jax
import jax.numpy as jnp
from jax import lax
from jax.experimental import pallas as pl
from jax.experimental.pallas import tpu as pltpu
from jax.experimental.pallas import tpu_sc as plsc

BATCH = 16384
EMB = 64
NC = 2
NS = 16
NW = NC * NS
B_PER_W = BATCH // NW
K = 128
C = B_PER_W // K


def _gather_body(uidx_hbm, midx_hbm, utab_hbm, mtab_hbm, uout_hbm, mout_hbm,
                 uidx_v, midx_v, urows_v, mrows_v, sem):
    wid = lax.axis_index("s") * NC + lax.axis_index("c")
    base = wid * B_PER_W
    pltpu.sync_copy(uidx_hbm.at[wid], uidx_v)
    pltpu.sync_copy(midx_hbm.at[wid], midx_v)
    copies = []
    for j in range(C):
        copies.append(pltpu.async_copy(
            utab_hbm.at[uidx_v.at[j]], urows_v.at[pl.ds(j * K, K)], sem))
        copies.append(pltpu.async_copy(
            mtab_hbm.at[midx_v.at[j]], mrows_v.at[pl.ds(j * K, K)], sem))
    for cp in copies:
        cp.wait()
    pltpu.sync_copy(urows_v, uout_hbm.at[pl.ds(base, B_PER_W)])
    pltpu.sync_copy(mrows_v, mout_hbm.at[pl.ds(base, B_PER_W)])


def _sc_gather(uidx, midx, user_table, movie_table):
    mesh = plsc.VectorSubcoreMesh(core_axis_name="c", subcore_axis_name="s")
    dt = user_table.dtype
    run = functools.partial(
        pl.kernel,
        mesh=mesh,
        compiler_params=pltpu.CompilerParams(use_tc_tiling_on_sc=False),
        out_type=(
            jax.ShapeDtypeStruct((BATCH, EMB), dt),
            jax.ShapeDtypeStruct((BATCH, EMB), dt),
        ),
        scratch_types=[
            pltpu.VMEM((C, K), jnp.int32),
            pltpu.VMEM((C, K), jnp.int32),
            pltpu.VMEM((B_PER_W, EMB), dt),
            pltpu.VMEM((B_PER_W, EMB), dt),
            pltpu.SemaphoreType.DMA,
        ],
    )(_gather_body)
    return run(uidx, midx, user_table, movie_table)


def _transpose_body(t_ref, o_ref):
    x = t_ref[...].astype(jnp.bfloat16)
    eye = jnp.eye(EMB, dtype=jnp.bfloat16)
    o_ref[...] = lax.dot_general(x, eye, (((0,), (0,)), ((), ())),
                                 preferred_element_type=jnp.float32
                                 ).astype(jnp.bfloat16)


def _tc_transpose(table_t, rows, bc):
    grid = (pl.cdiv(rows, bc),)
    return pl.pallas_call(
        _transpose_body,
        grid=grid,
        in_specs=[pl.BlockSpec((EMB, bc), lambda i: (0, i))],
        out_specs=pl.BlockSpec((bc, EMB), lambda i: (i, 0)),
        out_shape=jax.ShapeDtypeStruct((rows, EMB), jnp.bfloat16),
    )(table_t)


def _mlp_body(ue_ref, me_ref, w1_ref, b1_ref, w2_ref, b2_ref, o_ref):
    w1 = w1_ref[...].astype(jnp.bfloat16)
    h = lax.dot_general(ue_ref[...], w1[:, :EMB], (((1,), (1,)), ((), ())),
                        preferred_element_type=jnp.float32)
    h = h + lax.dot_general(me_ref[...], w1[:, EMB:], (((1,), (1,)), ((), ())),
                            preferred_element_type=jnp.float32)
    h = jnp.maximum(h + b1_ref[...], 0.0)
    o = jnp.sum(h * w2_ref[...], axis=1, keepdims=True)
    o_ref[...] = o + b2_ref[0, 0]


def _tc_mlp(ue, me, W1, b1, W2, b2):
    br = 2048
    grid = (BATCH // br,)
    return pl.pallas_call(
        _mlp_body,
        grid=grid,
        in_specs=[
            pl.BlockSpec((br, EMB), lambda i: (i, 0)),
            pl.BlockSpec((br, EMB), lambda i: (i, 0)),
            pl.BlockSpec((128, 2 * EMB), lambda i: (0, 0)),
            pl.BlockSpec((1, 128), lambda i: (0, 0)),
            pl.BlockSpec((1, 128), lambda i: (0, 0)),
            pl.BlockSpec((1, 1), lambda i: (0, 0)),
        ],
        out_specs=pl.BlockSpec((br, 1), lambda i: (i, 0)),
        out_shape=jax.ShapeDtypeStruct((BATCH, 1), jnp.float32),
    )(ue, me, W1, b1.reshape(1, 128), W2, b2.reshape(1, 1))


def kernel(user, movie, user_table, movie_table, W1, b1, W2, b2):
    uidx = user.astype(jnp.int32).reshape(NW, C, K)
    midx = movie.astype(jnp.int32).reshape(NW, C, K)
    ut = _tc_transpose(user_table.T, user_table.shape[0], 2048)
    mt = _tc_transpose(movie_table.T, movie_table.shape[0], 2048)
    ue, me = _sc_gather(uidx, midx, ut, mt)
    out = _tc_mlp(ue, me, W1, b1, W2, b2)
    return out[:, 0]


# speedup vs baseline: 1.3995x; 1.3995x over previous
import functools

import jax
import jax.numpy as jnp
from jax import lax
from jax.experimental import pallas as pl
from jax.experimental.pallas import tpu as pltpu
from jax.experimental.pallas import tpu_sc as plsc

BATCH = 16384
EMB = 64
NC = 2
NS = 16
NW = NC * NS
B_PER_W = BATCH // NW
K = 128
CHUNK = 256
ROWS_PER_UNIT = 4


def _gather_body(uidx_hbm, midx_hbm, utab_hbm, mtab_hbm, uout_hbm, mout_hbm,
                 idx_v, rows_v, sem):
    wid = lax.axis_index("s") * NC + lax.axis_index("c")
    base = wid * B_PER_W
    for tab_hbm, out_hbm, ih in ((utab_hbm, uout_hbm, uidx_hbm),
                                 (mtab_hbm, mout_hbm, midx_hbm)):
        pltpu.sync_copy(ih.at[pl.ds(base, B_PER_W)], idx_v)
        for c in range(B_PER_W // CHUNK):
            copies = []
            for k in range(CHUNK // K):
                j = c * CHUNK + k * K
                copies.append(pltpu.async_copy(
                    tab_hbm.at[idx_v.at[pl.ds(j, K)]],
                    rows_v.at[pl.ds(k * K, K)], sem))
            for cp in copies:
                cp.wait()
            pltpu.sync_copy(rows_v, out_hbm.at[pl.ds(base + c * CHUNK, CHUNK)])


def _sc_gather(uidx, midx, utab3, mtab3):
    mesh = plsc.VectorSubcoreMesh(core_axis_name="c", subcore_axis_name="s")
    run = functools.partial(
        pl.kernel,
        mesh=mesh,
        compiler_params=pltpu.CompilerParams(use_tc_tiling_on_sc=True),
        out_type=(
            jax.ShapeDtypeStruct((BATCH, 2, 128), jnp.float32),
            jax.ShapeDtypeStruct((BATCH, 2, 128), jnp.float32),
        ),
        scratch_types=[
            pltpu.VMEM((B_PER_W,), jnp.int32),
            pltpu.VMEM((CHUNK, 2, 128), jnp.float32),
            pltpu.SemaphoreType.DMA,
        ],
    )(_gather_body)
    return run(uidx, midx, utab3, mtab3)


def _mlp_body(ue_ref, me_ref, uk_ref, mk_ref, w1_ref, b1_ref, w2_ref, b2_ref,
              o_ref):
    w1 = w1_ref[...]
    br = ue_ref.shape[0]
    h = jnp.zeros((br, 128), jnp.float32)
    for x_ref, k_ref, woff in ((ue_ref, uk_ref, 0), (me_ref, mk_ref, EMB)):
        wh = w1[:, woff:woff + EMB]
        for k in range(ROWS_PER_UNIT):
            x = x_ref[:, k // 2, (k % 2) * EMB:(k % 2 + 1) * EMB]
            hk = lax.dot_general(x, wh, (((1,), (1,)), ((), ())),
                                 preferred_element_type=jnp.float32)
            sel = (k_ref[...] == k).astype(jnp.float32)
            h = h + hk * sel
    h = jnp.maximum(h + b1_ref[...], 0.0)
    o = jnp.sum(h * w2_ref[...], axis=1, keepdims=True)
    o_ref[...] = o + b2_ref[0, 0]


def _tc_mlp(ue3, me3, uk, mk, W1, b1, W2, b2):
    br = 2048
    grid = (BATCH // br,)
    return pl.pallas_call(
        _mlp_body,
        grid=grid,
        in_specs=[
            pl.BlockSpec((br, 2, 128), lambda i: (i, 0, 0)),
            pl.BlockSpec((br, 2, 128), lambda i: (i, 0, 0)),
            pl.BlockSpec((br, 1), lambda i: (i, 0)),
            pl.BlockSpec((br, 1), lambda i: (i, 0)),
            pl.BlockSpec((128, 2 * EMB), lambda i: (0, 0)),
            pl.BlockSpec((1, 128), lambda i: (0, 0)),
            pl.BlockSpec((1, 128), lambda i: (0, 0)),
            pl.BlockSpec((1, 1), lambda i: (0, 0)),
        ],
        out_specs=pl.BlockSpec((br, 1), lambda i: (i, 0)),
        out_shape=jax.ShapeDtypeStruct((BATCH, 1), jnp.float32),
    )(ue3, me3, uk, mk, W1, b1.reshape(1, 128), W2, b2.reshape(1, 1))


def kernel(user, movie, user_table, movie_table, W1, b1, W2, b2):
    user = user.astype(jnp.int32)
    movie = movie.astype(jnp.int32)
    utab3 = user_table.reshape(-1, 2, 128)
    mtab3 = movie_table.reshape(-1, 2, 128)
    ue3, me3 = _sc_gather(user // ROWS_PER_UNIT, movie // ROWS_PER_UNIT,
                          utab3, mtab3)
    out = _tc_mlp(ue3, me3,
                  (user % ROWS_PER_UNIT).reshape(-1, 1),
                  (movie % ROWS_PER_UNIT).reshape(-1, 1),
                  W1, b1, W2, b2)
    return out[:, 0]


# speedup vs baseline: 1.5875x; 1.1343x over previous
import functools

import jax
import jax.numpy as jnp
from jax import lax
from jax.experimental import pallas as pl
from jax.experimental.pallas import tpu as pltpu
from jax.experimental.pallas import tpu_sc as plsc

BATCH = 16384
EMB = 64
NC = 2
NS = 16
NW = NC * NS
B_PER_W = BATCH // NW
K = 128
CHUNK = 256
ROWS_PER_UNIT = 2


def _gather_body(uidx_hbm, midx_hbm, utab_hbm, mtab_hbm, uout_hbm, mout_hbm,
                 idx_v, rows_v, sem):
    wid = lax.axis_index("s") * NC + lax.axis_index("c")
    base = wid * B_PER_W
    for tab_hbm, out_hbm, ih in ((utab_hbm, uout_hbm, uidx_hbm),
                                 (mtab_hbm, mout_hbm, midx_hbm)):
        pltpu.sync_copy(ih.at[pl.ds(base, B_PER_W)], idx_v)
        for c in range(B_PER_W // CHUNK):
            copies = []
            for k in range(CHUNK // K):
                j = c * CHUNK + k * K
                copies.append(pltpu.async_copy(
                    tab_hbm.at[idx_v.at[pl.ds(j, K)]],
                    rows_v.at[pl.ds(k * K, K)], sem))
            for cp in copies:
                cp.wait()
            pltpu.sync_copy(rows_v, out_hbm.at[pl.ds(base + c * CHUNK, CHUNK)])


def _sc_gather(uidx, midx, utab3, mtab3):
    mesh = plsc.VectorSubcoreMesh(core_axis_name="c", subcore_axis_name="s")
    run = functools.partial(
        pl.kernel,
        mesh=mesh,
        compiler_params=pltpu.CompilerParams(use_tc_tiling_on_sc=True),
        out_type=(
            jax.ShapeDtypeStruct((BATCH, 128), jnp.float32),
            jax.ShapeDtypeStruct((BATCH, 128), jnp.float32),
        ),
        scratch_types=[
            pltpu.VMEM((B_PER_W,), jnp.int32),
            pltpu.VMEM((CHUNK, 128), jnp.float32),
            pltpu.SemaphoreType.DMA,
        ],
    )(_gather_body)
    return run(uidx, midx, utab3, mtab3)


def _mlp_body(ue_ref, me_ref, uk_ref, mk_ref, w1_ref, b1_ref, w2_ref, b2_ref,
              o_ref):
    w1 = w1_ref[...]
    br = ue_ref.shape[0]
    h = jnp.zeros((br, 128), jnp.float32)
    for x_ref, k_ref, woff in ((ue_ref, uk_ref, 0), (me_ref, mk_ref, EMB)):
        wh = w1[:, woff:woff + EMB]
        for k in range(ROWS_PER_UNIT):
            x = x_ref[:, k * EMB:(k + 1) * EMB]
            hk = lax.dot_general(x, wh, (((1,), (1,)), ((), ())),
                                 preferred_element_type=jnp.float32)
            sel = (k_ref[...] == k).astype(jnp.float32)
            h = h + hk * sel
    h = jnp.maximum(h + b1_ref[...], 0.0)
    o = jnp.sum(h * w2_ref[...], axis=1, keepdims=True)
    o_ref[...] = o + b2_ref[0, 0]


def _tc_mlp(ue3, me3, uk, mk, W1, b1, W2, b2):
    br = 2048
    grid = (BATCH // br,)
    return pl.pallas_call(
        _mlp_body,
        grid=grid,
        in_specs=[
            pl.BlockSpec((br, 128), lambda i: (i, 0)),
            pl.BlockSpec((br, 128), lambda i: (i, 0)),
            pl.BlockSpec((br, 1), lambda i: (i, 0)),
            pl.BlockSpec((br, 1), lambda i: (i, 0)),
            pl.BlockSpec((128, 2 * EMB), lambda i: (0, 0)),
            pl.BlockSpec((1, 128), lambda i: (0, 0)),
            pl.BlockSpec((1, 128), lambda i: (0, 0)),
            pl.BlockSpec((1, 1), lambda i: (0, 0)),
        ],
        out_specs=pl.BlockSpec((br, 1), lambda i: (i, 0)),
        out_shape=jax.ShapeDtypeStruct((BATCH, 1), jnp.float32),
    )(ue3, me3, uk, mk, W1, b1.reshape(1, 128), W2, b2.reshape(1, 1))


def kernel(user, movie, user_table, movie_table, W1, b1, W2, b2):
    user = user.astype(jnp.int32)
    movie = movie.astype(jnp.int32)
    utab3 = user_table.reshape(-1, 128)
    mtab3 = movie_table.reshape(-1, 128)
    ue3, me3 = _sc_gather(user // ROWS_PER_UNIT, movie // ROWS_PER_UNIT,
                          utab3, mtab3)
    out = _tc_mlp(ue3, me3,
                  (user % ROWS_PER_UNIT).reshape(-1, 1),
                  (movie % ROWS_PER_UNIT).reshape(-1, 1),
                  W1, b1, W2, b2)
    return out[:, 0]


# speedup vs baseline: 1.6122x; 1.0155x over previous
import functools

import jax
import jax.numpy as jnp
from jax import lax
from jax.experimental import pallas as pl
from jax.experimental.pallas import tpu as pltpu
from jax.experimental.pallas import tpu_sc as plsc

BATCH = 16384
EMB = 64
NC = 2
NS = 16
NW = NC * NS
B_PER_W = BATCH // NW
K = 128
CHUNK = 256
ROWS_PER_UNIT = 2


def _gather_body(uidx_hbm, midx_hbm, utab_hbm, mtab_hbm, uout_hbm, mout_hbm,
                 idx_v, rows_v, sem):
    wid = lax.axis_index("s") * NC + lax.axis_index("c")
    base = wid * B_PER_W
    for tab_hbm, out_hbm, ih in ((utab_hbm, uout_hbm, uidx_hbm),
                                 (mtab_hbm, mout_hbm, midx_hbm)):
        pltpu.sync_copy(ih.at[pl.ds(base, B_PER_W)], idx_v)
        for c in range(B_PER_W // CHUNK):
            copies = []
            for k in range(CHUNK // K):
                j = c * CHUNK + k * K
                copies.append(pltpu.async_copy(
                    tab_hbm.at[idx_v.at[pl.ds(j, K)]],
                    rows_v.at[pl.ds(k * K, K)], sem))
            for cp in copies:
                cp.wait()
            pltpu.sync_copy(rows_v, out_hbm.at[pl.ds(base + c * CHUNK, CHUNK)])


def _transpose_body(t_ref, o_ref):
    x = t_ref[...]
    eye = jnp.eye(EMB, dtype=jnp.float32)
    xt = lax.dot_general(x, eye, (((0,), (0,)), ((), ())),
                         preferred_element_type=jnp.float32)
    xt2 = xt.reshape(xt.shape[0] // 2, 2, EMB)
    o_ref[...] = jnp.concatenate([xt2[:, 0, :], xt2[:, 1, :]], axis=1)


def _tc_transpose(table_t, bc):
    rows = table_t.shape[1]
    grid = (pl.cdiv(rows, bc),)
    return pl.pallas_call(
        _transpose_body,
        grid=grid,
        in_specs=[pl.BlockSpec((EMB, bc), lambda i: (0, i))],
        out_specs=pl.BlockSpec((bc // 2, 128), lambda i: (i, 0)),
        out_shape=jax.ShapeDtypeStruct((rows // 2, 128), jnp.float32),
    )(table_t)


def _sc_gather(uidx, midx, utab3, mtab3):
    mesh = plsc.VectorSubcoreMesh(core_axis_name="c", subcore_axis_name="s")
    run = functools.partial(
        pl.kernel,
        mesh=mesh,
        compiler_params=pltpu.CompilerParams(use_tc_tiling_on_sc=True),
        out_type=(
            jax.ShapeDtypeStruct((BATCH, 128), jnp.float32),
            jax.ShapeDtypeStruct((BATCH, 128), jnp.float32),
        ),
        scratch_types=[
            pltpu.VMEM((B_PER_W,), jnp.int32),
            pltpu.VMEM((CHUNK, 128), jnp.float32),
            pltpu.SemaphoreType.DMA,
        ],
    )(_gather_body)
    return run(uidx, midx, utab3, mtab3)


def _mlp_body(ue_ref, me_ref, uk_ref, mk_ref, w1_ref, b1_ref, w2_ref, b2_ref,
              o_ref):
    w1 = w1_ref[...]
    br = ue_ref.shape[0]
    h = jnp.zeros((br, 128), jnp.float32)
    for x_ref, k_ref, woff in ((ue_ref, uk_ref, 0), (me_ref, mk_ref, EMB)):
        wh = w1[:, woff:woff + EMB]
        for k in range(ROWS_PER_UNIT):
            x = x_ref[:, k * EMB:(k + 1) * EMB]
            hk = lax.dot_general(x, wh, (((1,), (1,)), ((), ())),
                                 preferred_element_type=jnp.float32)
            sel = (k_ref[...] == k).astype(jnp.float32)
            h = h + hk * sel
    h = jnp.maximum(h + b1_ref[...], 0.0)
    o = jnp.sum(h * w2_ref[...], axis=1, keepdims=True)
    o_ref[...] = o + b2_ref[0, 0]


def _tc_mlp(ue3, me3, uk, mk, W1, b1, W2, b2):
    br = 2048
    grid = (BATCH // br,)
    return pl.pallas_call(
        _mlp_body,
        grid=grid,
        in_specs=[
            pl.BlockSpec((br, 128), lambda i: (i, 0)),
            pl.BlockSpec((br, 128), lambda i: (i, 0)),
            pl.BlockSpec((br, 1), lambda i: (i, 0)),
            pl.BlockSpec((br, 1), lambda i: (i, 0)),
            pl.BlockSpec((128, 2 * EMB), lambda i: (0, 0)),
            pl.BlockSpec((1, 128), lambda i: (0, 0)),
            pl.BlockSpec((1, 128), lambda i: (0, 0)),
            pl.BlockSpec((1, 1), lambda i: (0, 0)),
        ],
        out_specs=pl.BlockSpec((br, 1), lambda i: (i, 0)),
        out_shape=jax.ShapeDtypeStruct((BATCH, 1), jnp.float32),
    )(ue3, me3, uk, mk, W1, b1.reshape(1, 128), W2, b2.reshape(1, 1))


def kernel(user, movie, user_table, movie_table, W1, b1, W2, b2):
    user = user.astype(jnp.int32)
    movie = movie.astype(jnp.int32)
    utab3 = _tc_transpose(user_table.T, 2048)
    mtab3 = _tc_transpose(movie_table.T, 2048)
    ue3, me3 = _sc_gather(user // ROWS_PER_UNIT, movie // ROWS_PER_UNIT,
                          utab3, mtab3)
    out = _tc_mlp(ue3, me3,
                  (user % ROWS_PER_UNIT).reshape(-1, 1),
                  (movie % ROWS_PER_UNIT).reshape(-1, 1),
                  W1, b1, W2, b2)
    return out[:, 0]


# speedup vs baseline: 2.0435x; 1.2675x over previous
import functools

import jax
import jax.numpy as jnp
from jax import lax
from jax.experimental import pallas as pl
from jax.experimental.pallas import tpu as pltpu
from jax.experimental.pallas import tpu_sc as plsc

BATCH = 16384
EMB = 64
NC = 2
NS = 16
NW = NC * NS
B_PER_W = BATCH // NW
K = 128
CHUNK = 256
ROWS_PER_UNIT = 2


def _gather_body(uidx_hbm, midx_hbm, utab_hbm, mtab_hbm, uout_hbm, mout_hbm,
                 idx_v, rows_v, sem):
    wid = lax.axis_index("s") * NC + lax.axis_index("c")
    base = wid * B_PER_W
    for tab_hbm, out_hbm, ih in ((utab_hbm, uout_hbm, uidx_hbm),
                                 (mtab_hbm, mout_hbm, midx_hbm)):
        pltpu.sync_copy(ih.at[pl.ds(base, B_PER_W)], idx_v)
        for c in range(B_PER_W // CHUNK):
            copies = []
            for k in range(CHUNK // K):
                j = c * CHUNK + k * K
                copies.append(pltpu.async_copy(
                    tab_hbm.at[idx_v.at[pl.ds(j, K)]],
                    rows_v.at[pl.ds(k * K, K)], sem))
            for cp in copies:
                cp.wait()
            pltpu.sync_copy(rows_v, out_hbm.at[pl.ds(base + c * CHUNK, CHUNK)])


def _transpose_body(t_ref, o_ref):
    x = t_ref[...]
    eye = jnp.eye(EMB, dtype=jnp.float32)
    xt = lax.dot_general(x, eye, (((0,), (0,)), ((), ())),
                         preferred_element_type=jnp.float32)
    xt2 = xt.reshape(xt.shape[0] // 2, 2, EMB)
    o_ref[...] = jnp.concatenate([xt2[:, 0, :], xt2[:, 1, :]], axis=1)


def _tc_transpose(table_t, bc):
    rows = table_t.shape[1]
    grid = (pl.cdiv(rows, bc),)
    return pl.pallas_call(
        _transpose_body,
        grid=grid,
        in_specs=[pl.BlockSpec((EMB, bc), lambda i: (0, i))],
        out_specs=pl.BlockSpec((bc // 2, 128), lambda i: (i, 0)),
        out_shape=jax.ShapeDtypeStruct((rows // 2, 128), jnp.float32),
    )(table_t)


def _sc_gather(uidx, midx, utab3, mtab3):
    mesh = plsc.VectorSubcoreMesh(core_axis_name="c", subcore_axis_name="s")
    run = functools.partial(
        pl.kernel,
        mesh=mesh,
        compiler_params=pltpu.CompilerParams(use_tc_tiling_on_sc=True),
        out_type=(
            jax.ShapeDtypeStruct((BATCH, 128), jnp.float32),
            jax.ShapeDtypeStruct((BATCH, 128), jnp.float32),
        ),
        scratch_types=[
            pltpu.VMEM((B_PER_W,), jnp.int32),
            pltpu.VMEM((CHUNK, 128), jnp.float32),
            pltpu.SemaphoreType.DMA,
        ],
    )(_gather_body)
    return run(uidx, midx, utab3, mtab3)


def _mlp_body(ue_ref, me_ref, uk_ref, mk_ref, w1_ref, b1_ref, w2_ref, b2_ref,
              o_ref):
    w1 = w1_ref[...]
    br = ue_ref.shape[0]
    h = jnp.zeros((br, 128), jnp.float32)
    for x_ref, k_ref, woff in ((ue_ref, uk_ref, 0), (me_ref, mk_ref, EMB)):
        wh = w1[:, woff:woff + EMB]
        for k in range(ROWS_PER_UNIT):
            x = x_ref[:, k * EMB:(k + 1) * EMB]
            hk = lax.dot_general(x, wh, (((1,), (1,)), ((), ())),
                                 preferred_element_type=jnp.float32)
            sel = (k_ref[...] == k).astype(jnp.float32)
            h = h + hk * sel
    h = jnp.maximum(h + b1_ref[...], 0.0)
    o = jnp.sum(h * w2_ref[...], axis=1, keepdims=True)
    o_ref[...] = o + b2_ref[0, 0]


def _tc_mlp(ue3, me3, uk, mk, W1, b1, W2, b2):
    br = 2048
    grid = (BATCH // br,)
    return pl.pallas_call(
        _mlp_body,
        grid=grid,
        in_specs=[
            pl.BlockSpec((br, 128), lambda i: (i, 0)),
            pl.BlockSpec((br, 128), lambda i: (i, 0)),
            pl.BlockSpec((br, 1), lambda i: (i, 0)),
            pl.BlockSpec((br, 1), lambda i: (i, 0)),
            pl.BlockSpec((128, 2 * EMB), lambda i: (0, 0)),
            pl.BlockSpec((1, 128), lambda i: (0, 0)),
            pl.BlockSpec((1, 128), lambda i: (0, 0)),
            pl.BlockSpec((1, 1), lambda i: (0, 0)),
        ],
        out_specs=pl.BlockSpec((br, 1), lambda i: (i, 0)),
        out_shape=jax.ShapeDtypeStruct((BATCH, 1), jnp.float32),
    )(ue3, me3, uk, mk, W1, b1.reshape(1, 128), W2, b2.reshape(1, 1))


def kernel(user, movie, user_table, movie_table, W1, b1, W2, b2):
    user = user.astype(jnp.int32)
    movie = movie.astype(jnp.int32)
    utab3 = _tc_transpose(user_table.T, 8192)
    mtab3 = _tc_transpose(movie_table.T, 8192)
    ue3, me3 = _sc_gather(user // ROWS_PER_UNIT, movie // ROWS_PER_UNIT,
                          utab3, mtab3)
    out = _tc_mlp(ue3, me3,
                  (user % ROWS_PER_UNIT).reshape(-1, 1),
                  (movie % ROWS_PER_UNIT).reshape(-1, 1),
                  W1, b1, W2, b2)
    return out[:, 0]


# speedup vs baseline: 2.0957x; 1.0255x over previous
import functools

import jax
import jax.numpy as jnp
from jax import lax
from jax.experimental import pallas as pl
from jax.experimental.pallas import tpu as pltpu
from jax.experimental.pallas import tpu_sc as plsc

BATCH = 16384
EMB = 64
NC = 2
NS = 16
NW = NC * NS
B_PER_W = BATCH // NW
K = 128
CHUNK = 256
ROWS_PER_UNIT = 2


def _gather_body(uidx_hbm, midx_hbm, utab_hbm, mtab_hbm, uout_hbm, mout_hbm,
                 idx_v, rows_v, sem):
    wid = lax.axis_index("s") * NC + lax.axis_index("c")
    base = wid * B_PER_W
    for tab_hbm, out_hbm, ih in ((utab_hbm, uout_hbm, uidx_hbm),
                                 (mtab_hbm, mout_hbm, midx_hbm)):
        pltpu.sync_copy(ih.at[pl.ds(base, B_PER_W)], idx_v)
        for c in range(B_PER_W // CHUNK):
            copies = []
            for k in range(CHUNK // K):
                j = c * CHUNK + k * K
                copies.append(pltpu.async_copy(
                    tab_hbm.at[idx_v.at[pl.ds(j, K)]],
                    rows_v.at[pl.ds(k * K, K)], sem))
            for cp in copies:
                cp.wait()
            pltpu.sync_copy(rows_v, out_hbm.at[pl.ds(base + c * CHUNK, CHUNK)])


def _transpose_body(t_ref, o_ref):
    x = t_ref[...]
    eye = jnp.eye(EMB, dtype=jnp.float32)
    xt = lax.dot_general(x, eye, (((0,), (0,)), ((), ())),
                         preferred_element_type=jnp.float32)
    xt2 = xt.reshape(xt.shape[0] // 2, 2, EMB)
    o_ref[...] = jnp.concatenate([xt2[:, 0, :], xt2[:, 1, :]], axis=1)


def _tc_transpose(table_t, bc):
    rows = table_t.shape[1]
    grid = (pl.cdiv(rows, bc),)
    return pl.pallas_call(
        _transpose_body,
        grid=grid,
        in_specs=[pl.BlockSpec((EMB, bc), lambda i: (0, i))],
        out_specs=pl.BlockSpec((bc // 2, 128), lambda i: (i, 0)),
        out_shape=jax.ShapeDtypeStruct((rows // 2, 128), jnp.float32),
    )(table_t)


def _sc_gather(uidx, midx, utab3, mtab3):
    mesh = plsc.VectorSubcoreMesh(core_axis_name="c", subcore_axis_name="s")
    run = functools.partial(
        pl.kernel,
        mesh=mesh,
        compiler_params=pltpu.CompilerParams(use_tc_tiling_on_sc=True),
        out_type=(
            jax.ShapeDtypeStruct((BATCH, 128), jnp.float32),
            jax.ShapeDtypeStruct((BATCH, 128), jnp.float32),
        ),
        scratch_types=[
            pltpu.VMEM((B_PER_W,), jnp.int32),
            pltpu.VMEM((CHUNK, 128), jnp.float32),
            pltpu.SemaphoreType.DMA,
        ],
    )(_gather_body)
    return run(uidx, midx, utab3, mtab3)


def _mlp_body(ue_ref, me_ref, uk_ref, mk_ref, w1_ref, b1_ref, w2_ref, b2_ref,
              o_ref):
    w1 = w1_ref[...]
    br = ue_ref.shape[0]
    h = jnp.zeros((br, 128), jnp.float32)
    for x_ref, k_ref, woff in ((ue_ref, uk_ref, 0), (me_ref, mk_ref, EMB)):
        wh = w1[:, woff:woff + EMB]
        for k in range(ROWS_PER_UNIT):
            x = x_ref[:, k * EMB:(k + 1) * EMB]
            hk = lax.dot_general(x, wh, (((1,), (1,)), ((), ())),
                                 preferred_element_type=jnp.float32)
            sel = (k_ref[...] == k).astype(jnp.float32)
            h = h + hk * sel
    h = jnp.maximum(h + b1_ref[...], 0.0)
    o = jnp.sum(h * w2_ref[...], axis=1, keepdims=True)
    o_ref[...] = o + b2_ref[0, 0]


def _tc_mlp(ue3, me3, uk, mk, W1, b1, W2, b2):
    br = 2048
    grid = (BATCH // br,)
    return pl.pallas_call(
        _mlp_body,
        grid=grid,
        in_specs=[
            pl.BlockSpec((br, 128), lambda i: (i, 0)),
            pl.BlockSpec((br, 128), lambda i: (i, 0)),
            pl.BlockSpec((br, 1), lambda i: (i, 0)),
            pl.BlockSpec((br, 1), lambda i: (i, 0)),
            pl.BlockSpec((128, 2 * EMB), lambda i: (0, 0)),
            pl.BlockSpec((1, 128), lambda i: (0, 0)),
            pl.BlockSpec((1, 128), lambda i: (0, 0)),
            pl.BlockSpec((1, 1), lambda i: (0, 0)),
        ],
        out_specs=pl.BlockSpec((br, 1), lambda i: (i, 0)),
        out_shape=jax.ShapeDtypeStruct((BATCH, 1), jnp.float32),
    )(ue3, me3, uk, mk, W1, b1.reshape(1, 128), W2, b2.reshape(1, 1))


def kernel(user, movie, user_table, movie_table, W1, b1, W2, b2):
    user = user.astype(jnp.int32)
    movie = movie.astype(jnp.int32)
    utab3 = _tc_transpose(user_table.T, 16384)
    mtab3 = _tc_transpose(movie_table.T, 12800)
    ue3, me3 = _sc_gather(user // ROWS_PER_UNIT, movie // ROWS_PER_UNIT,
                          utab3, mtab3)
    out = _tc_mlp(ue3, me3,
                  (user % ROWS_PER_UNIT).reshape(-1, 1),
                  (movie % ROWS_PER_UNIT).reshape(-1, 1),
                  W1, b1, W2, b2)
    return out[:, 0]
